# baseline (speedup 1.0000x reference)
import functools

import jax
import jax.numpy as jnp
from jax import lax
from jax.experimental import pallas as pl
from jax.experimental.pallas import tpu as pltpu
from jax.experimental.pallas import tpu_sc as plsc

MAXZ = 84
NRP = 24
NATOMS = 1048576

NC = 2
NS = 16
NW = NC * NS

PER_W = NATOMS // NW
CHUNK = 2048
NCHUNK = PER_W // CHUNK


def _gather_sc(Z, p):
    mesh = plsc.VectorSubcoreMesh(core_axis_name="c", subcore_axis_name="s")

    @functools.partial(
        pl.kernel,
        mesh=mesh,
        out_type=jax.ShapeDtypeStruct((NATOMS, NRP), jnp.float32),
        scratch_types=[
            pltpu.VMEM_SHARED((MAXZ, NRP), jnp.float32),
            pltpu.VMEM((CHUNK,), jnp.int32),
            pltpu.VMEM((CHUNK, NRP), jnp.float32),
            pltpu.SemaphoreType.DMA,
        ],
        compiler_params=pltpu.CompilerParams(use_tc_tiling_on_sc=False),
    )
    def k(z_hbm, p_hbm, out_hbm, table_v, idx_v, rows_v, sem):
        sid = lax.axis_index("s")
        wid = sid * NC + lax.axis_index("c")
        base = wid * PER_W

        @pl.when(sid == 0)
        def _stage():
            pltpu.sync_copy(p_hbm, table_v)

        plsc.subcore_barrier()

        def body(c, carry):
            off = base + c * CHUNK
            pltpu.sync_copy(z_hbm.at[pl.ds(off, CHUNK)], idx_v)
            pltpu.async_copy(p_hbm.at[idx_v], rows_v, sem).wait()
            pltpu.sync_copy(rows_v, out_hbm.at[pl.ds(off, CHUNK), :])
            return carry

        lax.fori_loop(0, NCHUNK, body, 0)

    return k(Z, p)


def kernel(Z, p, alpha, chi):
    Z32 = Z.astype(jnp.int32)
    gathered = _gather_sc(Z32, p)
    return (gathered, alpha, chi)


# speedup vs baseline: 1.8502x; 1.8502x over previous
import functools

import jax
import jax.numpy as jnp
from jax import lax
from jax.experimental import pallas as pl
from jax.experimental.pallas import tpu as pltpu
from jax.experimental.pallas import tpu_sc as plsc

MAXZ = 84
NRP = 24
NATOMS = 1048576

NC = 2
NS = 16
NW = NC * NS

PER_W = NATOMS // NW
CHUNK = 2048
NCHUNK = PER_W // CHUNK


def _gather_sc(Z, p):
    mesh = plsc.VectorSubcoreMesh(core_axis_name="c", subcore_axis_name="s")

    @functools.partial(
        pl.kernel,
        mesh=mesh,
        out_type=jax.ShapeDtypeStruct((NATOMS, NRP), jnp.float32),
        scratch_types=[
            pltpu.VMEM_SHARED((MAXZ, NRP), jnp.float32),
            pltpu.VMEM((2, CHUNK), jnp.int32),
            pltpu.VMEM((2, CHUNK, NRP), jnp.float32),
            pltpu.SemaphoreType.DMA((2,)),
            pltpu.SemaphoreType.DMA((2,)),
            pltpu.SemaphoreType.DMA((2,)),
        ],
        compiler_params=pltpu.CompilerParams(use_tc_tiling_on_sc=False),
    )
    def k(z_hbm, p_hbm, out_hbm, table_v, idx_v, rows_v, isem, gsem, osem):
        sid = lax.axis_index("s")
        wid = sid * NC + lax.axis_index("c")
        base = wid * PER_W

        @pl.when(sid == 0)
        def _stage():
            pltpu.sync_copy(p_hbm, table_v)

        plsc.subcore_barrier()

        idx_cp = [None, None]
        gat_cp = [None, None]
        out_cp = [None, None]

        def start_idx(c):
            s = c % 2
            idx_cp[s] = pltpu.async_copy(
                z_hbm.at[pl.ds(base + c * CHUNK, CHUNK)], idx_v.at[s], isem.at[s]
            )

        def start_gather(c):
            s = c % 2
            gat_cp[s] = pltpu.async_copy(
                table_v.at[idx_v.at[s]], rows_v.at[s], gsem.at[s]
            )

        def start_write(c):
            s = c % 2
            out_cp[s] = pltpu.async_copy(
                rows_v.at[s], out_hbm.at[pl.ds(base + c * CHUNK, CHUNK), :],
                osem.at[s],
            )

        start_idx(0)
        start_idx(1)
        idx_cp[0].wait()
        start_gather(0)

        for c in range(1, NCHUNK):
            s = c % 2
            idx_cp[s].wait()
            if c >= 2:
                out_cp[s].wait()
            start_gather(c)
            gat_cp[1 - s].wait()
            start_write(c - 1)
            if c + 1 < NCHUNK:
                start_idx(c + 1)

        last = NCHUNK - 1
        s = last % 2
        gat_cp[s].wait()
        start_write(last)
        out_cp[1 - s].wait()
        out_cp[s].wait()

    return k(Z, p)


def kernel(Z, p, alpha, chi):
    Z32 = Z.astype(jnp.int32)
    gathered = _gather_sc(Z32, p)
    return (gathered, alpha, chi)
